# baseline (speedup 1.0000x reference)
import functools

import jax
import jax.numpy as jnp
import numpy as np
from jax import lax
from jax.experimental import pallas as pl
from jax.experimental.pallas import tpu as pltpu
from jax.experimental.pallas import tpu_sc as plsc

E = 32
G = 4
TOPK = 700
P = 3136
H = 56
KPAD = 704
WLANES = 512

_INTERPRET = False


def _heads_body(x_ref, wb_ref, bb_ref, wo_ref, bo_ref, we_ref, be_ref,
                ww_ref, bw_ref, obj_ref, enc_ref, wmap_ref):
    feat = jnp.maximum(
        jnp.dot(x_ref[...], wb_ref[...], preferred_element_type=jnp.float32)
        + bb_ref[...], 0.0)
    obj_ref[...] = (
        jnp.dot(feat, wo_ref[...], preferred_element_type=jnp.float32)
        + bo_ref[...])
    enc_ref[...] = (
        jnp.dot(feat, we_ref[...], preferred_element_type=jnp.float32)
        + be_ref[...])
    wmap_ref[...] = (
        jnp.dot(feat, ww_ref[...], preferred_element_type=jnp.float32)
        + bw_ref[...])


def _topk_gather_body(vrow_ref, vcol_ref, wmap_ref, wsel_ref, scores_ref):
    s_row = jax.nn.sigmoid(vrow_ref[...])
    s_col = jax.nn.sigmoid(vcol_ref[...])
    rank = jnp.zeros((1, P), jnp.int32)
    jblk = 448
    for b in range(P // jblk):
        sj = s_col[b * jblk:(b + 1) * jblk, :]
        jidx = b * jblk + jax.lax.broadcasted_iota(jnp.int32, (jblk, P), 0)
        iidx = jax.lax.broadcasted_iota(jnp.int32, (jblk, P), 1)
        gt = sj > s_row
        eq = (sj == s_row) & (jidx < iidx)
        rank = rank + jnp.sum((gt | eq).astype(jnp.int32), axis=0,
                              keepdims=True)
    kblk = 176
    for b in range(KPAD // kblk):
        kidx = b * kblk + jax.lax.broadcasted_iota(jnp.int32, (kblk, P), 0)
        oneh = (kidx == rank).astype(jnp.float32)
        wsel_ref[b * kblk:(b + 1) * kblk, :] = jnp.dot(
            oneh, wmap_ref[...], preferred_element_type=jnp.float32)
        scores_ref[b * kblk:(b + 1) * kblk, :] = jnp.sum(
            oneh * s_row, axis=1, keepdims=True)


_LOG2E = 1.4426950408889634


def _decode_body(wsel_ref, enc_ref, m_ref):
    acc = None
    for g in range(G):
        wg = wsel_ref[:, 128 * g:128 * g + E + 1]
        z = jnp.dot(wg, enc_ref[...],
                    preferred_element_type=jnp.float32)
        q = 1.0 + jnp.exp2(z * (-_LOG2E))
        acc = q if acc is None else acc * q
    m = 1.0 / acc
    for h in range(H):
        m_ref[:, h, :] = m[:, 128 * h:128 * (h + 1)]


def _upsample_body(m_ref, ut_ref, u_ref, out_ref, kb):
    a = m_ref[...].reshape(kb * H, 128)
    x1 = jnp.dot(a, ut_ref[...],
                 preferred_element_type=jnp.float32)
    for k in range(kb):
        out_ref[0, k] = jnp.dot(u_ref[...], x1[k * H:(k + 1) * H, :],
                                preferred_element_type=jnp.float32)


_NW = 32
_OUT_W = 4 * H
_CHUNKS = _OUT_W // 16


def _wpass_tables():
    o = np.arange(_OUT_W)
    pos = (o + 0.5) / 4.0 - 0.5
    lo = np.floor(pos).astype(np.int64)
    w1 = (pos - lo).astype(np.float32)
    li0 = np.clip(lo, 0, H - 1).astype(np.int32)
    li1 = np.clip(lo + 1, 0, H - 1).astype(np.int32)
    return li0, li1, (1.0 - w1).astype(np.float32), w1


def _sc_upsample(m56):
    li0, li1, fw0, fw1 = _wpass_tables()
    mesh = plsc.VectorSubcoreMesh(core_axis_name="c", subcore_axis_name="s")

    @functools.partial(
        pl.kernel,
        out_type=jax.ShapeDtypeStruct((TOPK, 392, 128), jnp.float32),
        mesh=mesh,
        compiler_params=pltpu.CompilerParams(needs_layout_passes=False),
        scratch_types=[
            pltpu.VMEM((H, 128), jnp.float32),
            pltpu.VMEM((H, _OUT_W), jnp.float32),
            pltpu.VMEM((2, 392, 128), jnp.float32),
            pltpu.VMEM((_OUT_W,), jnp.int32),
            pltpu.VMEM((_OUT_W,), jnp.int32),
            pltpu.VMEM((_OUT_W,), jnp.float32),
            pltpu.VMEM((_OUT_W,), jnp.float32),
            pltpu.SemaphoreType.DMA,
        ],
    )
    def sc_up(m_hbm, li0_hbm, li1_hbm, fw0_hbm, fw1_hbm, out_hbm,
              inb, x1b, outb, li0v, li1v, fw0v, fw1v, sem_out):
        nc = 2
        wid = lax.axis_index("s") * nc + lax.axis_index("c")
        n_t = (TOPK - wid + _NW - 1) // _NW
        pltpu.sync_copy(li0_hbm, li0v)
        pltpu.sync_copy(li1_hbm, li1v)
        pltpu.sync_copy(fw0_hbm, fw0v)
        pltpu.sync_copy(fw1_hbm, fw1v)

        def store_out(s, row, c, vec):
            flat = row * _OUT_W + 16 * c
            outb[s, flat // 128, pl.ds(flat % 128, 16)] = vec

        def step(t, carry):
            s = lax.rem(t, 2)
            i = wid + _NW * t

            @pl.when(t >= 2)
            def _wait_prev():
                pltpu.make_async_copy(outb.at[s], out_hbm.at[i],
                                      sem_out).wait()

            pltpu.sync_copy(m_hbm.at[i], inb)

            def wpass(h, carry2):
                hv = jnp.full((16,), h, jnp.int32)
                for c in range(_CHUNKS):
                    ds = pl.ds(16 * c, 16)
                    a = plsc.load_gather(inb, [hv, li0v[ds]])
                    b = plsc.load_gather(inb, [hv, li1v[ds]])
                    x1b[h, ds] = fw0v[ds] * a + fw1v[ds] * b
                return carry2

            lax.fori_loop(0, H, wpass, 0)

            for c in range(_CHUNKS):
                ds = pl.ds(16 * c, 16)
                v0 = x1b[0, ds]
                store_out(s, 0, c, v0)
                store_out(s, 1, c, v0)
                v1 = x1b[H - 1, ds]
                store_out(s, 4 * H - 2, c, v1)
                store_out(s, 4 * H - 1, c, v1)

            def hpass(w, carry2):
                for c in range(_CHUNKS):
                    ds = pl.ds(16 * c, 16)
                    va = x1b[w, ds]
                    vb = x1b[w + 1, ds]
                    store_out(s, 4 * w + 2, c, 0.875 * va + 0.125 * vb)
                    store_out(s, 4 * w + 3, c, 0.625 * va + 0.375 * vb)
                    store_out(s, 4 * w + 4, c, 0.375 * va + 0.625 * vb)
                    store_out(s, 4 * w + 5, c, 0.125 * va + 0.875 * vb)
                return carry2

            lax.fori_loop(0, H - 1, hpass, 0)

            pltpu.make_async_copy(outb.at[s], out_hbm.at[i], sem_out).start()
            return carry

        lax.fori_loop(0, n_t, step, 0)
        pltpu.make_async_copy(outb.at[0], out_hbm.at[0], sem_out).wait()
        pltpu.make_async_copy(outb.at[1], out_hbm.at[0], sem_out).wait()

    return sc_up(m56, jnp.asarray(li0), jnp.asarray(li1),
                 jnp.asarray(fw0), jnp.asarray(fw1))


def _upsample_matrix():
    o = np.arange(4 * H)
    pos = (o + 0.5) / 4.0 - 0.5
    lo = np.floor(pos).astype(np.int64)
    w = (pos - lo).astype(np.float32)
    u = np.zeros((4 * H, H), np.float32)
    for i in range(4 * H):
        l = min(max(int(lo[i]), 0), H - 1)
        h = min(max(int(lo[i]) + 1, 0), H - 1)
        u[i, l] += 1.0 - w[i]
        u[i, h] += w[i]
    return u


def kernel(image, Wb, bb, Wo, bo, We, be, Ww, bw):
    f32 = jnp.float32
    x = image.reshape(3, H, 4, H, 4).transpose(1, 3, 0, 2, 4).reshape(P, 48)
    wb_t = Wb.reshape(96, 48).T
    bb2 = bb.reshape(1, 96)
    wo_t = jnp.zeros((96, 128), f32).at[:, 0].set(Wo[0])
    bo2 = jnp.zeros((1, 128), f32).at[0, 0].set(bo[0])
    we_t = We.T
    be2 = be.reshape(1, E)
    lane = (128 * (np.arange((E + 1) * G) // (E + 1))
            + np.arange((E + 1) * G) % (E + 1))
    ww_t = jnp.zeros((96, WLANES), f32).at[:, lane].set(Ww.T)
    bw2 = jnp.zeros((1, WLANES), f32).at[0, lane].set(bw)

    obj_full, enc, wmap = pl.pallas_call(
        _heads_body,
        out_shape=(
            jax.ShapeDtypeStruct((P, 128), f32),
            jax.ShapeDtypeStruct((P, E), f32),
            jax.ShapeDtypeStruct((P, WLANES), f32),
        ),
        interpret=_INTERPRET,
    )(x, wb_t, bb2, wo_t, bo2, we_t, be2, ww_t, bw2)

    obj_col = obj_full[:, :1]
    obj_row = obj_col.reshape(1, P)

    wsel, scores = pl.pallas_call(
        _topk_gather_body,
        out_shape=(
            jax.ShapeDtypeStruct((KPAD, WLANES), f32),
            jax.ShapeDtypeStruct((KPAD, 1), f32),
        ),
        interpret=_INTERPRET,
    )(obj_row, obj_col, wmap)

    enc_t = enc.T
    enc_aug = jnp.zeros((E + 1, H, 128), f32)
    enc_aug = enc_aug.at[:E, :, :H].set(enc_t.reshape(E, H, H))
    enc_aug = enc_aug.at[E, :, :H].set(1.0)
    enc_aug = enc_aug.reshape(E + 1, H * 128)
    kb3 = 64
    m56 = pl.pallas_call(
        _decode_body,
        grid=(KPAD // kb3,),
        in_specs=[
            pl.BlockSpec((kb3, WLANES), lambda i: (i, 0)),
            pl.BlockSpec((E + 1, H * 128), lambda i: (0, 0)),
        ],
        out_specs=pl.BlockSpec((kb3, H, 128), lambda i: (i, 0, 0)),
        out_shape=jax.ShapeDtypeStruct((TOPK, H, 128), f32),
        interpret=_INTERPRET,
    )(wsel, enc_aug)

    masks = _sc_upsample(m56).reshape(1, TOPK, 4 * H, 4 * H)

    obj_logits = obj_col.reshape(1, 1, H, H)
    return obj_logits, masks, scores[:TOPK, 0].reshape(1, TOPK)


# speedup vs baseline: 2.1149x; 2.1149x over previous
import functools

import jax
import jax.numpy as jnp
import numpy as np
from jax import lax
from jax.experimental import pallas as pl
from jax.experimental.pallas import tpu as pltpu
from jax.experimental.pallas import tpu_sc as plsc

E = 32
G = 4
TOPK = 700
P = 3136
H = 56
KPAD = 704
WLANES = 512

_INTERPRET = False


def _heads_body(x_ref, wb_ref, bb_ref, wo_ref, bo_ref, we_ref, be_ref,
                ww_ref, bw_ref, obj_ref, enc_ref, wmap_ref):
    feat = jnp.maximum(
        jnp.dot(x_ref[...], wb_ref[...], preferred_element_type=jnp.float32)
        + bb_ref[...], 0.0)
    obj_ref[...] = (
        jnp.dot(feat, wo_ref[...], preferred_element_type=jnp.float32)
        + bo_ref[...])
    enc_ref[...] = (
        jnp.dot(feat, we_ref[...], preferred_element_type=jnp.float32)
        + be_ref[...])
    wmap_ref[...] = (
        jnp.dot(feat, ww_ref[...], preferred_element_type=jnp.float32)
        + bw_ref[...])


def _topk_gather_body(vrow_ref, vcol_ref, wmap_ref, wsel_ref, scores_ref):
    s_row = jax.nn.sigmoid(vrow_ref[...])
    s_col = jax.nn.sigmoid(vcol_ref[...])
    rank = jnp.zeros((1, P), jnp.int32)
    jblk = 448
    for b in range(P // jblk):
        sj = s_col[b * jblk:(b + 1) * jblk, :]
        jidx = b * jblk + jax.lax.broadcasted_iota(jnp.int32, (jblk, P), 0)
        iidx = jax.lax.broadcasted_iota(jnp.int32, (jblk, P), 1)
        gt = sj > s_row
        eq = (sj == s_row) & (jidx < iidx)
        rank = rank + jnp.sum((gt | eq).astype(jnp.int32), axis=0,
                              keepdims=True)
    kblk = 176
    for b in range(KPAD // kblk):
        kidx = b * kblk + jax.lax.broadcasted_iota(jnp.int32, (kblk, P), 0)
        oneh = (kidx == rank).astype(jnp.float32)
        wsel_ref[b * kblk:(b + 1) * kblk, :] = jnp.dot(
            oneh, wmap_ref[...], preferred_element_type=jnp.float32)
        scores_ref[b * kblk:(b + 1) * kblk, :] = jnp.sum(
            oneh * s_row, axis=1, keepdims=True)


_LOG2E = 1.4426950408889634


def _decode_body(wsel_ref, enc_ref, m_ref):
    acc = None
    for g in range(G):
        wg = wsel_ref[:, 128 * g:128 * g + E + 1]
        z = jnp.dot(wg, enc_ref[...],
                    preferred_element_type=jnp.float32)
        q = 1.0 + jnp.exp2(z * (-_LOG2E))
        acc = q if acc is None else acc * q
    m = 1.0 / acc
    for h in range(H):
        m_ref[:, h, :] = m[:, 128 * h:128 * (h + 1)]


_NBUF = 4


def _upsample_body(m_ref, ut_ref, u_ref, out_ref, scratch, sems, kb, nsteps):
    i = pl.program_id(0)
    slot = lax.rem(i, _NBUF)

    @pl.when(i >= _NBUF)
    def _wait_slot():
        pltpu.make_async_copy(
            scratch.at[slot], out_ref.at[0, pl.ds((i - _NBUF) * kb, kb)],
            sems.at[slot]).wait()

    a = m_ref[...].reshape(kb * H, 128)
    x1 = jnp.dot(a, ut_ref[...],
                 preferred_element_type=jnp.float32)
    for k in range(kb):
        scratch[slot, k] = jnp.dot(u_ref[...], x1[k * H:(k + 1) * H, :],
                                   preferred_element_type=jnp.float32)
    pltpu.make_async_copy(
        scratch.at[slot], out_ref.at[0, pl.ds(i * kb, kb)],
        sems.at[slot]).start()

    @pl.when(i == nsteps - 1)
    def _drain():
        for j in range(_NBUF):
            sj = (nsteps - _NBUF + j) % _NBUF
            pltpu.make_async_copy(
                scratch.at[sj],
                out_ref.at[0, pl.ds((nsteps - _NBUF + j) * kb, kb)],
                sems.at[sj]).wait()


_NW = 32
_OUT_W = 4 * H
_CHUNKS = _OUT_W // 16


def _wpass_tables():
    o = np.arange(_OUT_W)
    pos = (o + 0.5) / 4.0 - 0.5
    lo = np.floor(pos).astype(np.int64)
    w1 = (pos - lo).astype(np.float32)
    li0 = np.clip(lo, 0, H - 1).astype(np.int32)
    li1 = np.clip(lo + 1, 0, H - 1).astype(np.int32)
    return li0, li1, (1.0 - w1).astype(np.float32), w1


def _sc_upsample(m56):
    li0, li1, fw0, fw1 = _wpass_tables()
    mesh = plsc.VectorSubcoreMesh(core_axis_name="c", subcore_axis_name="s")

    @functools.partial(
        pl.kernel,
        out_type=jax.ShapeDtypeStruct((TOPK, 392, 128), jnp.float32),
        mesh=mesh,
        compiler_params=pltpu.CompilerParams(needs_layout_passes=False),
        scratch_types=[
            pltpu.VMEM((H, 128), jnp.float32),
            pltpu.VMEM((H, _OUT_W), jnp.float32),
            pltpu.VMEM((2, 392, 128), jnp.float32),
            pltpu.VMEM((_OUT_W,), jnp.int32),
            pltpu.VMEM((_OUT_W,), jnp.int32),
            pltpu.VMEM((_OUT_W,), jnp.float32),
            pltpu.VMEM((_OUT_W,), jnp.float32),
            pltpu.SemaphoreType.DMA,
        ],
    )
    def sc_up(m_hbm, li0_hbm, li1_hbm, fw0_hbm, fw1_hbm, out_hbm,
              inb, x1b, outb, li0v, li1v, fw0v, fw1v, sem_out):
        nc = 2
        wid = lax.axis_index("s") * nc + lax.axis_index("c")
        n_t = (TOPK - wid + _NW - 1) // _NW
        pltpu.sync_copy(li0_hbm, li0v)
        pltpu.sync_copy(li1_hbm, li1v)
        pltpu.sync_copy(fw0_hbm, fw0v)
        pltpu.sync_copy(fw1_hbm, fw1v)

        def store_out(s, row, c, vec):
            flat = row * _OUT_W + 16 * c
            outb[s, flat // 128, pl.ds(flat % 128, 16)] = vec

        def step(t, carry):
            s = lax.rem(t, 2)
            i = wid + _NW * t

            @pl.when(t >= 2)
            def _wait_prev():
                pltpu.make_async_copy(outb.at[s], out_hbm.at[i],
                                      sem_out).wait()

            pltpu.sync_copy(m_hbm.at[i], inb)

            def wpass(h, carry2):
                hv = jnp.full((16,), h, jnp.int32)
                for c in range(_CHUNKS):
                    ds = pl.ds(16 * c, 16)
                    a = plsc.load_gather(inb, [hv, li0v[ds]])
                    b = plsc.load_gather(inb, [hv, li1v[ds]])
                    x1b[h, ds] = fw0v[ds] * a + fw1v[ds] * b
                return carry2

            lax.fori_loop(0, H, wpass, 0)

            for c in range(_CHUNKS):
                ds = pl.ds(16 * c, 16)
                v0 = x1b[0, ds]
                store_out(s, 0, c, v0)
                store_out(s, 1, c, v0)
                v1 = x1b[H - 1, ds]
                store_out(s, 4 * H - 2, c, v1)
                store_out(s, 4 * H - 1, c, v1)

            def hpass(w, carry2):
                for c in range(_CHUNKS):
                    ds = pl.ds(16 * c, 16)
                    va = x1b[w, ds]
                    vb = x1b[w + 1, ds]
                    store_out(s, 4 * w + 2, c, 0.875 * va + 0.125 * vb)
                    store_out(s, 4 * w + 3, c, 0.625 * va + 0.375 * vb)
                    store_out(s, 4 * w + 4, c, 0.375 * va + 0.625 * vb)
                    store_out(s, 4 * w + 5, c, 0.125 * va + 0.875 * vb)
                return carry2

            lax.fori_loop(0, H - 1, hpass, 0)

            pltpu.make_async_copy(outb.at[s], out_hbm.at[i], sem_out).start()
            return carry

        lax.fori_loop(0, n_t, step, 0)
        pltpu.make_async_copy(outb.at[0], out_hbm.at[0], sem_out).wait()
        pltpu.make_async_copy(outb.at[1], out_hbm.at[0], sem_out).wait()

    return sc_up(m56, jnp.asarray(li0), jnp.asarray(li1),
                 jnp.asarray(fw0), jnp.asarray(fw1))


def _upsample_matrix():
    o = np.arange(4 * H)
    pos = (o + 0.5) / 4.0 - 0.5
    lo = np.floor(pos).astype(np.int64)
    w = (pos - lo).astype(np.float32)
    u = np.zeros((4 * H, H), np.float32)
    for i in range(4 * H):
        l = min(max(int(lo[i]), 0), H - 1)
        h = min(max(int(lo[i]) + 1, 0), H - 1)
        u[i, l] += 1.0 - w[i]
        u[i, h] += w[i]
    return u


def kernel(image, Wb, bb, Wo, bo, We, be, Ww, bw):
    f32 = jnp.float32
    x = image.reshape(3, H, 4, H, 4).transpose(1, 3, 0, 2, 4).reshape(P, 48)
    wb_t = Wb.reshape(96, 48).T
    bb2 = bb.reshape(1, 96)
    wo_t = jnp.zeros((96, 128), f32).at[:, 0].set(Wo[0])
    bo2 = jnp.zeros((1, 128), f32).at[0, 0].set(bo[0])
    we_t = We.T
    be2 = be.reshape(1, E)
    lane = (128 * (np.arange((E + 1) * G) // (E + 1))
            + np.arange((E + 1) * G) % (E + 1))
    ww_t = jnp.zeros((96, WLANES), f32).at[:, lane].set(Ww.T)
    bw2 = jnp.zeros((1, WLANES), f32).at[0, lane].set(bw)

    obj_full, enc, wmap = pl.pallas_call(
        _heads_body,
        out_shape=(
            jax.ShapeDtypeStruct((P, 128), f32),
            jax.ShapeDtypeStruct((P, E), f32),
            jax.ShapeDtypeStruct((P, WLANES), f32),
        ),
        interpret=_INTERPRET,
    )(x, wb_t, bb2, wo_t, bo2, we_t, be2, ww_t, bw2)

    obj_col = obj_full[:, :1]
    obj_row = obj_col.reshape(1, P)

    wsel, scores = pl.pallas_call(
        _topk_gather_body,
        out_shape=(
            jax.ShapeDtypeStruct((KPAD, WLANES), f32),
            jax.ShapeDtypeStruct((KPAD, 1), f32),
        ),
        interpret=_INTERPRET,
    )(obj_row, obj_col, wmap)

    enc_t = enc.T
    enc_aug = jnp.zeros((E + 1, H, 128), f32)
    enc_aug = enc_aug.at[:E, :, :H].set(enc_t.reshape(E, H, H))
    enc_aug = enc_aug.at[E, :, :H].set(1.0)
    enc_aug = enc_aug.reshape(E + 1, H * 128)
    kb3 = 64
    m56 = pl.pallas_call(
        _decode_body,
        grid=(KPAD // kb3,),
        in_specs=[
            pl.BlockSpec((kb3, WLANES), lambda i: (i, 0)),
            pl.BlockSpec((E + 1, H * 128), lambda i: (0, 0)),
        ],
        out_specs=pl.BlockSpec((kb3, H, 128), lambda i: (i, 0, 0)),
        out_shape=jax.ShapeDtypeStruct((TOPK, H, 128), f32),
        interpret=_INTERPRET,
    )(wsel, enc_aug)

    u = jnp.asarray(_upsample_matrix())
    ut128 = np.zeros((128, 4 * H), np.float32)
    ut128[:H, :] = _upsample_matrix().T
    ut = jnp.asarray(ut128)
    kb4 = 28
    nsteps = TOPK // kb4
    masks = pl.pallas_call(
        functools.partial(_upsample_body, kb=kb4, nsteps=nsteps),
        grid=(nsteps,),
        in_specs=[
            pl.BlockSpec((kb4, H, 128), lambda i: (i, 0, 0)),
            pl.BlockSpec((128, 4 * H), lambda i: (0, 0)),
            pl.BlockSpec((4 * H, H), lambda i: (0, 0)),
        ],
        out_specs=pl.BlockSpec(memory_space=pl.ANY),
        out_shape=jax.ShapeDtypeStruct((1, TOPK, 4 * H, 4 * H), f32),
        scratch_shapes=[
            pltpu.VMEM((_NBUF, kb4, 4 * H, 4 * H), f32),
            pltpu.SemaphoreType.DMA((_NBUF,)),
        ],
        interpret=_INTERPRET,
    )(m56, ut, u)

    obj_logits = obj_col.reshape(1, 1, H, H)
    return obj_logits, masks, scores[:TOPK, 0].reshape(1, TOPK)


# speedup vs baseline: 2.3539x; 1.1130x over previous
import functools

import jax
import jax.numpy as jnp
import numpy as np
from jax import lax
from jax.experimental import pallas as pl
from jax.experimental.pallas import tpu as pltpu
from jax.experimental.pallas import tpu_sc as plsc

E = 32
G = 4
TOPK = 700
P = 3136
H = 56
KPAD = 704
WLANES = 512

_INTERPRET = False


def _heads_body(x_ref, wb_ref, bb_ref, wo_ref, bo_ref, we_ref, be_ref,
                ww_ref, bw_ref, obj_ref, enc_ref, wmap_ref):
    feat = jnp.maximum(
        jnp.dot(x_ref[...], wb_ref[...], preferred_element_type=jnp.float32)
        + bb_ref[...], 0.0)
    obj_ref[...] = (
        jnp.dot(feat, wo_ref[...], preferred_element_type=jnp.float32)
        + bo_ref[...])
    enc_ref[...] = (
        jnp.dot(feat, we_ref[...], preferred_element_type=jnp.float32)
        + be_ref[...])
    wmap_ref[...] = (
        jnp.dot(feat, ww_ref[...], preferred_element_type=jnp.float32)
        + bw_ref[...])


def _topk_gather_body(vrow_ref, vcol_ref, wmap_ref, wsel_ref, scores_ref):
    s_row = jax.nn.sigmoid(vrow_ref[...])
    s_col = jax.nn.sigmoid(vcol_ref[...])
    rank = jnp.zeros((1, P), jnp.int32)
    jblk = 448
    for b in range(P // jblk):
        sj = s_col[b * jblk:(b + 1) * jblk, :]
        jidx = b * jblk + jax.lax.broadcasted_iota(jnp.int32, (jblk, P), 0)
        iidx = jax.lax.broadcasted_iota(jnp.int32, (jblk, P), 1)
        gt = sj > s_row
        eq = (sj == s_row) & (jidx < iidx)
        rank = rank + jnp.sum((gt | eq).astype(jnp.int32), axis=0,
                              keepdims=True)
    kblk = 176
    for b in range(KPAD // kblk):
        kidx = b * kblk + jax.lax.broadcasted_iota(jnp.int32, (kblk, P), 0)
        oneh = (kidx == rank).astype(jnp.float32)
        wsel_ref[b * kblk:(b + 1) * kblk, :] = jnp.dot(
            oneh, wmap_ref[...], preferred_element_type=jnp.float32)
        scores_ref[b * kblk:(b + 1) * kblk, :] = jnp.sum(
            oneh * s_row, axis=1, keepdims=True)


_LOG2E = 1.4426950408889634


def _decode_body(wsel_ref, enc_ref, m_ref):
    acc = None
    for g in range(G):
        wg = wsel_ref[:, 128 * g:128 * g + E + 1]
        z = jnp.dot(wg, enc_ref[...],
                    preferred_element_type=jnp.float32)
        q = 1.0 + jnp.exp2(z * (-_LOG2E))
        acc = q if acc is None else acc * q
    m = 1.0 / acc
    for h in range(H):
        m_ref[:, h, :] = m[:, 128 * h:128 * (h + 1)]


def _decup_body(wsel_ref, enc_ref, ut_ref, u_ref, out_ref, scratch, sems,
                kb, nsteps):
    i = pl.program_id(0)
    slot = lax.rem(i, 2)

    @pl.when(i >= 2)
    def _wait_slot():
        pltpu.make_async_copy(
            scratch.at[slot], out_ref.at[0, pl.ds((i - 2) * kb, kb)],
            sems.at[slot]).wait()

    acc = None
    for g in range(G):
        wg = wsel_ref[:, 128 * g:128 * g + E + 1]
        z = jnp.dot(wg, enc_ref[...],
                    preferred_element_type=jnp.float32)
        q = 1.0 + jnp.exp2(z * (-_LOG2E))
        acc = q if acc is None else acc * q
    m = 1.0 / acc
    a = m.reshape(kb * H, 128)
    x1 = jnp.dot(a, ut_ref[...],
                 preferred_element_type=jnp.float32)
    for k in range(kb):
        scratch[slot, k] = jnp.dot(u_ref[...], x1[k * H:(k + 1) * H, :],
                                   preferred_element_type=jnp.float32)

    @pl.when(i < nsteps - 1)
    def _start_full():
        pltpu.make_async_copy(
            scratch.at[slot], out_ref.at[0, pl.ds(i * kb, kb)],
            sems.at[slot]).start()

    @pl.when(i == nsteps - 1)
    def _last():
        tail = TOPK - (nsteps - 1) * kb
        pltpu.make_async_copy(
            scratch.at[0, pl.ds(0, tail)],
            out_ref.at[0, pl.ds((nsteps - 1) * kb, tail)],
            sems.at[0]).start()
        pltpu.make_async_copy(
            scratch.at[1], out_ref.at[0, pl.ds((nsteps - 2) * kb, kb)],
            sems.at[1]).wait()
        pltpu.make_async_copy(
            scratch.at[0, pl.ds(0, tail)],
            out_ref.at[0, pl.ds((nsteps - 1) * kb, tail)],
            sems.at[0]).wait()


_NBUF = 4


def _upsample_body(m_ref, ut_ref, u_ref, out_ref, scratch, sems, kb, nsteps):
    i = pl.program_id(0)
    slot = lax.rem(i, _NBUF)

    @pl.when(i >= _NBUF)
    def _wait_slot():
        pltpu.make_async_copy(
            scratch.at[slot], out_ref.at[0, pl.ds((i - _NBUF) * kb, kb)],
            sems.at[slot]).wait()

    a = m_ref[...].reshape(kb * H, 128)
    x1 = jnp.dot(a, ut_ref[...],
                 preferred_element_type=jnp.float32)
    for k in range(kb):
        scratch[slot, k] = jnp.dot(u_ref[...], x1[k * H:(k + 1) * H, :],
                                   preferred_element_type=jnp.float32)
    pltpu.make_async_copy(
        scratch.at[slot], out_ref.at[0, pl.ds(i * kb, kb)],
        sems.at[slot]).start()

    @pl.when(i == nsteps - 1)
    def _drain():
        for j in range(_NBUF):
            sj = (nsteps - _NBUF + j) % _NBUF
            pltpu.make_async_copy(
                scratch.at[sj],
                out_ref.at[0, pl.ds((nsteps - _NBUF + j) * kb, kb)],
                sems.at[sj]).wait()


_NW = 32
_OUT_W = 4 * H
_CHUNKS = _OUT_W // 16


def _wpass_tables():
    o = np.arange(_OUT_W)
    pos = (o + 0.5) / 4.0 - 0.5
    lo = np.floor(pos).astype(np.int64)
    w1 = (pos - lo).astype(np.float32)
    li0 = np.clip(lo, 0, H - 1).astype(np.int32)
    li1 = np.clip(lo + 1, 0, H - 1).astype(np.int32)
    return li0, li1, (1.0 - w1).astype(np.float32), w1


def _sc_upsample(m56):
    li0, li1, fw0, fw1 = _wpass_tables()
    mesh = plsc.VectorSubcoreMesh(core_axis_name="c", subcore_axis_name="s")

    @functools.partial(
        pl.kernel,
        out_type=jax.ShapeDtypeStruct((TOPK, 392, 128), jnp.float32),
        mesh=mesh,
        compiler_params=pltpu.CompilerParams(needs_layout_passes=False),
        scratch_types=[
            pltpu.VMEM((H, 128), jnp.float32),
            pltpu.VMEM((H, _OUT_W), jnp.float32),
            pltpu.VMEM((2, 392, 128), jnp.float32),
            pltpu.VMEM((_OUT_W,), jnp.int32),
            pltpu.VMEM((_OUT_W,), jnp.int32),
            pltpu.VMEM((_OUT_W,), jnp.float32),
            pltpu.VMEM((_OUT_W,), jnp.float32),
            pltpu.SemaphoreType.DMA,
        ],
    )
    def sc_up(m_hbm, li0_hbm, li1_hbm, fw0_hbm, fw1_hbm, out_hbm,
              inb, x1b, outb, li0v, li1v, fw0v, fw1v, sem_out):
        nc = 2
        wid = lax.axis_index("s") * nc + lax.axis_index("c")
        n_t = (TOPK - wid + _NW - 1) // _NW
        pltpu.sync_copy(li0_hbm, li0v)
        pltpu.sync_copy(li1_hbm, li1v)
        pltpu.sync_copy(fw0_hbm, fw0v)
        pltpu.sync_copy(fw1_hbm, fw1v)

        def store_out(s, row, c, vec):
            flat = row * _OUT_W + 16 * c
            outb[s, flat // 128, pl.ds(flat % 128, 16)] = vec

        def step(t, carry):
            s = lax.rem(t, 2)
            i = wid + _NW * t

            @pl.when(t >= 2)
            def _wait_prev():
                pltpu.make_async_copy(outb.at[s], out_hbm.at[i],
                                      sem_out).wait()

            pltpu.sync_copy(m_hbm.at[i], inb)

            def wpass(h, carry2):
                hv = jnp.full((16,), h, jnp.int32)
                for c in range(_CHUNKS):
                    ds = pl.ds(16 * c, 16)
                    a = plsc.load_gather(inb, [hv, li0v[ds]])
                    b = plsc.load_gather(inb, [hv, li1v[ds]])
                    x1b[h, ds] = fw0v[ds] * a + fw1v[ds] * b
                return carry2

            lax.fori_loop(0, H, wpass, 0)

            for c in range(_CHUNKS):
                ds = pl.ds(16 * c, 16)
                v0 = x1b[0, ds]
                store_out(s, 0, c, v0)
                store_out(s, 1, c, v0)
                v1 = x1b[H - 1, ds]
                store_out(s, 4 * H - 2, c, v1)
                store_out(s, 4 * H - 1, c, v1)

            def hpass(w, carry2):
                for c in range(_CHUNKS):
                    ds = pl.ds(16 * c, 16)
                    va = x1b[w, ds]
                    vb = x1b[w + 1, ds]
                    store_out(s, 4 * w + 2, c, 0.875 * va + 0.125 * vb)
                    store_out(s, 4 * w + 3, c, 0.625 * va + 0.375 * vb)
                    store_out(s, 4 * w + 4, c, 0.375 * va + 0.625 * vb)
                    store_out(s, 4 * w + 5, c, 0.125 * va + 0.875 * vb)
                return carry2

            lax.fori_loop(0, H - 1, hpass, 0)

            pltpu.make_async_copy(outb.at[s], out_hbm.at[i], sem_out).start()
            return carry

        lax.fori_loop(0, n_t, step, 0)
        pltpu.make_async_copy(outb.at[0], out_hbm.at[0], sem_out).wait()
        pltpu.make_async_copy(outb.at[1], out_hbm.at[0], sem_out).wait()

    return sc_up(m56, jnp.asarray(li0), jnp.asarray(li1),
                 jnp.asarray(fw0), jnp.asarray(fw1))


def _upsample_matrix():
    o = np.arange(4 * H)
    pos = (o + 0.5) / 4.0 - 0.5
    lo = np.floor(pos).astype(np.int64)
    w = (pos - lo).astype(np.float32)
    u = np.zeros((4 * H, H), np.float32)
    for i in range(4 * H):
        l = min(max(int(lo[i]), 0), H - 1)
        h = min(max(int(lo[i]) + 1, 0), H - 1)
        u[i, l] += 1.0 - w[i]
        u[i, h] += w[i]
    return u


def kernel(image, Wb, bb, Wo, bo, We, be, Ww, bw):
    f32 = jnp.float32
    x = image.reshape(3, H, 4, H, 4).transpose(1, 3, 0, 2, 4).reshape(P, 48)
    wb_t = Wb.reshape(96, 48).T
    bb2 = bb.reshape(1, 96)
    wo_t = jnp.zeros((96, 128), f32).at[:, 0].set(Wo[0])
    bo2 = jnp.zeros((1, 128), f32).at[0, 0].set(bo[0])
    we_t = We.T
    be2 = be.reshape(1, E)
    lane = (128 * (np.arange((E + 1) * G) // (E + 1))
            + np.arange((E + 1) * G) % (E + 1))
    ww_t = jnp.zeros((96, WLANES), f32).at[:, lane].set(Ww.T)
    bw2 = jnp.zeros((1, WLANES), f32).at[0, lane].set(bw)

    obj_full, enc, wmap = pl.pallas_call(
        _heads_body,
        out_shape=(
            jax.ShapeDtypeStruct((P, 128), f32),
            jax.ShapeDtypeStruct((P, E), f32),
            jax.ShapeDtypeStruct((P, WLANES), f32),
        ),
        interpret=_INTERPRET,
    )(x, wb_t, bb2, wo_t, bo2, we_t, be2, ww_t, bw2)

    obj_col = obj_full[:, :1]
    obj_row = obj_col.reshape(1, P)

    wsel, scores = pl.pallas_call(
        _topk_gather_body,
        out_shape=(
            jax.ShapeDtypeStruct((KPAD, WLANES), f32),
            jax.ShapeDtypeStruct((KPAD, 1), f32),
        ),
        interpret=_INTERPRET,
    )(obj_row, obj_col, wmap)

    enc_t = enc.T
    enc_aug = jnp.zeros((E + 1, H, 128), f32)
    enc_aug = enc_aug.at[:E, :, :H].set(enc_t.reshape(E, H, H))
    enc_aug = enc_aug.at[E, :, :H].set(1.0)
    enc_aug = enc_aug.reshape(E + 1, H * 128)
    u = jnp.asarray(_upsample_matrix())
    ut128 = np.zeros((128, 4 * H), np.float32)
    ut128[:H, :] = _upsample_matrix().T
    ut = jnp.asarray(ut128)
    kb = 56
    nsteps = 13
    masks = pl.pallas_call(
        functools.partial(_decup_body, kb=kb, nsteps=nsteps),
        grid=(nsteps,),
        in_specs=[
            pl.BlockSpec((kb, WLANES), lambda i: (i, 0)),
            pl.BlockSpec((E + 1, H * 128), lambda i: (0, 0)),
            pl.BlockSpec((128, 4 * H), lambda i: (0, 0)),
            pl.BlockSpec((4 * H, H), lambda i: (0, 0)),
        ],
        out_specs=pl.BlockSpec(memory_space=pl.ANY),
        out_shape=jax.ShapeDtypeStruct((1, TOPK, 4 * H, 4 * H), f32),
        scratch_shapes=[
            pltpu.VMEM((2, kb, 4 * H, 4 * H), f32),
            pltpu.SemaphoreType.DMA((2,)),
        ],
        interpret=_INTERPRET,
    )(wsel, enc_aug, ut, u)

    obj_logits = obj_col.reshape(1, 1, H, H)
    return obj_logits, masks, scores[:TOPK, 0].reshape(1, TOPK)
